# baseline (speedup 1.0000x reference)
import functools
import math

import jax
import jax.numpy as jnp
from jax import lax
from jax.experimental import pallas as pl
from jax.experimental.pallas import tpu as pltpu
from jax.experimental.pallas import tpu_sc as plsc

D_MODEL_ = 768
D_FF_ = 3072
N_EXP_ = 8
SEQ_ = 2048
TOPK_ = 2

TM = 256
T_MAX = SEQ_ * TOPK_ // TM + N_EXP_
T_PAD = T_MAX * TM

NW = 32
CH = SEQ_ // NW


def _erf(z):
    a1, a2, a3, a4, a5 = (0.254829592, -0.284496736, 1.421413741,
                          -1.453152027, 1.061405429)
    p = 0.3275911
    s = jnp.sign(z)
    x = jnp.abs(z)
    t = 1.0 / (1.0 + p * x)
    poly = t * (a1 + t * (a2 + t * (a3 + t * (a4 + t * a5))))
    return s * (1.0 - poly * jnp.exp(-x * x))


def _gelu_exact(h):
    return 0.5 * h * (1.0 + lax.erf(h * (1.0 / math.sqrt(2.0))))


def _route_body(x_ref, gw_ref, gb_ref,
                w1_ref, w2_ref, p1_ref, p2_ref, eid_ref, tot_ref):
    S, E = SEQ_, N_EXP_
    lg = lax.dot_general(
        x_ref[...].astype(jnp.bfloat16), gw_ref[...].astype(jnp.bfloat16),
        (((1,), (1,)), ((), ())),
        preferred_element_type=jnp.float32) + gb_ref[...]
    eio = lax.broadcasted_iota(jnp.int32, (S, E), 1)
    big = jnp.float32(1e30)
    m1 = jnp.max(lg, axis=1, keepdims=True)
    i1 = jnp.min(jnp.where(lg >= m1, eio, E), axis=1, keepdims=True)
    sel1 = eio == i1
    l2 = jnp.where(sel1, -big, lg)
    m2 = jnp.max(l2, axis=1, keepdims=True)
    i2 = jnp.min(jnp.where(l2 >= m2, eio, E), axis=1, keepdims=True)
    sel2 = eio == i2
    sel = sel1 | sel2
    ml = jnp.where(sel, lg, -big)
    pe = jnp.exp(ml - jnp.max(ml, axis=1, keepdims=True))
    w = pe / jnp.sum(pe, axis=1, keepdims=True)
    w1c = jnp.sum(jnp.where(sel1, w, 0.0), axis=1, keepdims=True)
    w2c = jnp.sum(jnp.where(sel2, w, 0.0), axis=1, keepdims=True)
    w1_ref[...] = jnp.broadcast_to(w1c, (S, 128))
    w2_ref[...] = jnp.broadcast_to(w2c, (S, 128))

    maskf = sel.astype(jnp.float32)
    rio = lax.broadcasted_iota(jnp.int32, (S, S), 0)
    cio = lax.broadcasted_iota(jnp.int32, (S, S), 1)
    ltri = (cio <= rio).astype(jnp.bfloat16)
    r_inc = lax.dot_general(
        ltri, maskf.astype(jnp.bfloat16), (((1,), (0,)), ((), ())),
        preferred_element_type=jnp.float32)
    r_exc = r_inc - maskf
    cnt = jnp.max(r_inc, axis=0, keepdims=True)
    inv_tm = jnp.float32(1.0 / TM)
    tiles = jnp.floor((cnt + (TM - 1)) * inv_tm)
    s8 = (lax.broadcasted_iota(jnp.int32, (E, E), 0)
          < lax.broadcasted_iota(jnp.int32, (E, E), 1)).astype(jnp.bfloat16)
    start_t = lax.dot_general(
        tiles.astype(jnp.bfloat16), s8, (((1,), (0,)), ((), ())),
        preferred_element_type=jnp.float32)
    pos = start_t * TM + r_exc
    p1_ref[...] = jnp.sum(jnp.where(sel1, pos, 0.0), axis=1,
                          keepdims=True).astype(jnp.int32)
    p2_ref[...] = jnp.sum(jnp.where(sel2, pos, 0.0), axis=1,
                          keepdims=True).astype(jnp.int32)

    end_t = start_t + tiles
    tio = lax.broadcasted_iota(jnp.int32, (32, E), 0).astype(jnp.float32)
    ended = jnp.sum((tio >= end_t).astype(jnp.float32), axis=1,
                    keepdims=True)
    eid_ref[...] = jnp.minimum(ended, float(N_EXP_ - 1)).astype(jnp.int32)
    tot_ref[...] = jnp.sum(tiles, axis=1, keepdims=True).astype(jnp.int32)


def _route(x2, gate_w, gb2):
    outs = pl.pallas_call(
        _route_body,
        grid=(1,),
        in_specs=[
            pl.BlockSpec((SEQ_, D_MODEL_), lambda i: (0, 0)),
            pl.BlockSpec((N_EXP_, D_MODEL_), lambda i: (0, 0)),
            pl.BlockSpec((1, N_EXP_), lambda i: (0, 0)),
        ],
        out_specs=[
            pl.BlockSpec((SEQ_, 128), lambda i: (0, 0)),
            pl.BlockSpec((SEQ_, 128), lambda i: (0, 0)),
            pl.BlockSpec((SEQ_, 1), lambda i: (0, 0)),
            pl.BlockSpec((SEQ_, 1), lambda i: (0, 0)),
            pl.BlockSpec((32, 1), lambda i: (0, 0)),
            pl.BlockSpec((1, 1), lambda i: (0, 0)),
        ],
        out_shape=[
            jax.ShapeDtypeStruct((SEQ_, 128), jnp.float32),
            jax.ShapeDtypeStruct((SEQ_, 128), jnp.float32),
            jax.ShapeDtypeStruct((SEQ_, 1), jnp.int32),
            jax.ShapeDtypeStruct((SEQ_, 1), jnp.int32),
            jax.ShapeDtypeStruct((32, 1), jnp.int32),
            jax.ShapeDtypeStruct((1, 1), jnp.int32),
        ],
    )(x2, gate_w, gb2)
    return outs


XW = D_MODEL_ // 2 + 128


def _dispatch_body(xw1_hbm, xw2_hbm, p1_hbm, p2_hbm,
                   xg_hbm,
                   p1v, p2v, xv1, xv2, sem):
    cid = lax.axis_index("c")
    sid = lax.axis_index("s")
    wid = cid * 16 + sid
    base = wid * CH
    lds = [
        pltpu.async_copy(p1_hbm.at[pl.ds(base, CH)], p1v, sem),
        pltpu.async_copy(p2_hbm.at[pl.ds(base, CH)], p2v, sem),
        pltpu.async_copy(xw1_hbm.at[pl.ds(base, CH)], xv1, sem),
        pltpu.async_copy(xw2_hbm.at[pl.ds(base, CH)], xv2, sem),
    ]
    for c in lds:
        c.wait()
    s1 = pltpu.async_copy(xv1, xg_hbm.at[p1v], sem)
    s2 = pltpu.async_copy(xv2, xg_hbm.at[p2v], sem)
    s1.wait()
    s2.wait()


def _dispatch(xw1, xw2, pos1, pos2):
    mesh = plsc.VectorSubcoreMesh(core_axis_name="c", subcore_axis_name="s")
    f = functools.partial(
        pl.kernel,
        mesh=mesh,
        out_type=jax.ShapeDtypeStruct((T_PAD, XW), jnp.int32),
        scratch_types=[
            pltpu.VMEM((CH,), jnp.int32),
            pltpu.VMEM((CH,), jnp.int32),
            pltpu.VMEM((CH, XW), jnp.int32),
            pltpu.VMEM((CH, XW), jnp.int32),
            pltpu.SemaphoreType.DMA,
        ],
    )(_dispatch_body)
    return f(xw1, xw2, pos1, pos2)


def _ffn_body(eid_ref, tot_ref, xg_ref, f1w_ref, f1b_ref, f2w_ref, f2b_ref,
              yg_ref):
    t = pl.program_id(0)

    @pl.when(t < tot_ref[0])
    def _():
        blk = lax.bitcast_convert_type(
            xg_ref[:, :D_MODEL_ // 2], jnp.uint32)
        lo = lax.bitcast_convert_type(blk << 16, jnp.float32)
        hi = lax.bitcast_convert_type(blk & jnp.uint32(0xFFFF0000),
                                      jnp.float32)
        xb = jnp.concatenate([lo, hi], axis=1).astype(jnp.bfloat16)
        h = lax.dot_general(
            xb, f1w_ref[0], (((1,), (1,)), ((), ())),
            preferred_element_type=jnp.float32) + f1b_ref[0]
        h = _gelu_exact(h)
        y = lax.dot_general(
            h.astype(jnp.bfloat16), f2w_ref[0], (((1,), (1,)), ((), ())),
            preferred_element_type=jnp.float32) + f2b_ref[0]
        wcol = lax.bitcast_convert_type(
            xg_ref[:, D_MODEL_ // 2:], jnp.float32)[:, 0:1]
        yg_ref[...] = y * wcol


def _ffn(eid, tot, xg, f1w_b, fc1_b3, f2w_b, fc2_b3):
    grid_spec = pltpu.PrefetchScalarGridSpec(
        num_scalar_prefetch=2,
        grid=(T_MAX,),
        in_specs=[
            pl.BlockSpec((TM, XW), lambda t, eid, tot: (t, 0)),
            pl.BlockSpec((1, D_FF_, D_MODEL_),
                         lambda t, eid, tot: (eid[t], 0, 0)),
            pl.BlockSpec((1, 1, D_FF_), lambda t, eid, tot: (eid[t], 0, 0)),
            pl.BlockSpec((1, D_MODEL_, D_FF_),
                         lambda t, eid, tot: (eid[t], 0, 0)),
            pl.BlockSpec((1, 1, D_MODEL_), lambda t, eid, tot: (eid[t], 0, 0)),
        ],
        out_specs=pl.BlockSpec((TM, D_MODEL_), lambda t, eid, tot: (t, 0)),
    )
    return pl.pallas_call(
        _ffn_body,
        grid_spec=grid_spec,
        out_shape=jax.ShapeDtypeStruct((T_PAD, D_MODEL_), jnp.float32),
    )(eid, tot, xg, f1w_b, fc1_b3, f2w_b, fc2_b3)


def _combine_body(yg_hbm, p1_hbm, p2_hbm, out_hbm, p1v, p2v, y1v, y2v, sem):
    cid = lax.axis_index("c")
    sid = lax.axis_index("s")
    wid = cid * 16 + sid
    base = wid * CH
    c1 = pltpu.async_copy(p1_hbm.at[pl.ds(base, CH)], p1v, sem)
    c2 = pltpu.async_copy(p2_hbm.at[pl.ds(base, CH)], p2v, sem)
    c1.wait()
    c2.wait()
    g1 = pltpu.async_copy(yg_hbm.at[p1v], y1v, sem)
    g2 = pltpu.async_copy(yg_hbm.at[p2v], y2v, sem)
    g1.wait()
    g2.wait()

    def row_body(r, carry):
        for j in range(D_MODEL_ // 16):
            sl = pl.ds(j * 16, 16)
            y1v[r, sl] = y1v[r, sl] + y2v[r, sl]
        return carry

    lax.fori_loop(0, CH, row_body, 0)
    pltpu.sync_copy(y1v, out_hbm.at[pl.ds(base, CH)])


def _combine(yg, pos1, pos2):
    mesh = plsc.VectorSubcoreMesh(core_axis_name="c", subcore_axis_name="s")
    f = functools.partial(
        pl.kernel,
        mesh=mesh,
        out_type=jax.ShapeDtypeStruct((SEQ_, D_MODEL_), jnp.float32),
        scratch_types=[
            pltpu.VMEM((CH,), jnp.int32),
            pltpu.VMEM((CH,), jnp.int32),
            pltpu.VMEM((CH, D_MODEL_), jnp.float32),
            pltpu.VMEM((CH, D_MODEL_), jnp.float32),
            pltpu.SemaphoreType.DMA,
        ],
    )(_combine_body)
    return f(yg, pos1, pos2)


def kernel(x, gate_w, gate_b, fc1_w, fc1_b, fc2_w, fc2_b):
    B, S, D = x.shape
    x2 = x.reshape(S, D)
    gb2 = gate_b.reshape(1, N_EXP_)
    w1, w2, pos1, pos2, eid, tot = _route(x2, gate_w, gb2)
    pos1 = pos1.reshape(S)
    pos2 = pos2.reshape(S)
    rbx = x2.astype(jnp.bfloat16).astype(jnp.float32)
    xbits = lax.bitcast_convert_type(rbx, jnp.uint32)
    xpack = lax.bitcast_convert_type(
        (xbits[:, :D // 2] >> 16) | xbits[:, D // 2:], jnp.int32)
    xw1 = jnp.concatenate(
        [xpack, lax.bitcast_convert_type(w1, jnp.int32)], axis=1)
    xw2 = jnp.concatenate(
        [xpack, lax.bitcast_convert_type(w2, jnp.int32)], axis=1)
    xg = _dispatch(xw1, xw2, pos1, pos2)
    yg = _ffn(eid.reshape(32), tot.reshape(1), xg,
              fc1_w.astype(jnp.bfloat16),
              fc1_b.reshape(N_EXP_, 1, D_FF_),
              fc2_w.astype(jnp.bfloat16),
              fc2_b.reshape(N_EXP_, 1, D_MODEL_))
    out = _combine(yg, pos1, pos2)
    return out.reshape(B, S, D)


# speedup vs baseline: 1.0635x; 1.0635x over previous
import functools
import math

import jax
import jax.numpy as jnp
from jax import lax
from jax.experimental import pallas as pl
from jax.experimental.pallas import tpu as pltpu
from jax.experimental.pallas import tpu_sc as plsc

D_MODEL_ = 768
D_FF_ = 3072
N_EXP_ = 8
SEQ_ = 2048
TOPK_ = 2

TM = 512
T_MAX = SEQ_ * TOPK_ // TM + N_EXP_
T_PAD = T_MAX * TM

NW = 32
CH = SEQ_ // NW


def _erf(z):
    a1, a2, a3, a4, a5 = (0.254829592, -0.284496736, 1.421413741,
                          -1.453152027, 1.061405429)
    p = 0.3275911
    s = jnp.sign(z)
    x = jnp.abs(z)
    t = 1.0 / (1.0 + p * x)
    poly = t * (a1 + t * (a2 + t * (a3 + t * (a4 + t * a5))))
    return s * (1.0 - poly * jnp.exp(-x * x))


def _gelu_exact(h):
    return 0.5 * h * (1.0 + lax.erf(h * (1.0 / math.sqrt(2.0))))


def _route_body(x_ref, gw_ref, gb_ref,
                w1_ref, w2_ref, p1_ref, p2_ref, eid_ref, tot_ref):
    S, E = SEQ_, N_EXP_
    lg = lax.dot_general(
        x_ref[...].astype(jnp.bfloat16), gw_ref[...].astype(jnp.bfloat16),
        (((1,), (1,)), ((), ())),
        preferred_element_type=jnp.float32) + gb_ref[...]
    eio = lax.broadcasted_iota(jnp.int32, (S, E), 1)
    big = jnp.float32(1e30)
    m1 = jnp.max(lg, axis=1, keepdims=True)
    i1 = jnp.min(jnp.where(lg >= m1, eio, E), axis=1, keepdims=True)
    sel1 = eio == i1
    l2 = jnp.where(sel1, -big, lg)
    m2 = jnp.max(l2, axis=1, keepdims=True)
    i2 = jnp.min(jnp.where(l2 >= m2, eio, E), axis=1, keepdims=True)
    sel2 = eio == i2
    sel = sel1 | sel2
    ml = jnp.where(sel, lg, -big)
    pe = jnp.exp(ml - jnp.max(ml, axis=1, keepdims=True))
    w = pe / jnp.sum(pe, axis=1, keepdims=True)
    w1c = jnp.sum(jnp.where(sel1, w, 0.0), axis=1, keepdims=True)
    w2c = jnp.sum(jnp.where(sel2, w, 0.0), axis=1, keepdims=True)
    w1_ref[...] = jnp.broadcast_to(w1c, (S, 128))
    w2_ref[...] = jnp.broadcast_to(w2c, (S, 128))

    maskf = sel.astype(jnp.float32)
    rio = lax.broadcasted_iota(jnp.int32, (S, S), 0)
    cio = lax.broadcasted_iota(jnp.int32, (S, S), 1)
    ltri = (cio <= rio).astype(jnp.bfloat16)
    r_inc = lax.dot_general(
        ltri, maskf.astype(jnp.bfloat16), (((1,), (0,)), ((), ())),
        preferred_element_type=jnp.float32)
    r_exc = r_inc - maskf
    cnt = jnp.max(r_inc, axis=0, keepdims=True)
    inv_tm = jnp.float32(1.0 / TM)
    tiles = jnp.floor((cnt + (TM - 1)) * inv_tm)
    s8 = (lax.broadcasted_iota(jnp.int32, (E, E), 0)
          < lax.broadcasted_iota(jnp.int32, (E, E), 1)).astype(jnp.bfloat16)
    start_t = lax.dot_general(
        tiles.astype(jnp.bfloat16), s8, (((1,), (0,)), ((), ())),
        preferred_element_type=jnp.float32)
    pos = start_t * TM + r_exc
    p1_ref[...] = jnp.sum(jnp.where(sel1, pos, 0.0), axis=1,
                          keepdims=True).astype(jnp.int32)
    p2_ref[...] = jnp.sum(jnp.where(sel2, pos, 0.0), axis=1,
                          keepdims=True).astype(jnp.int32)

    end_t = start_t + tiles
    tio = lax.broadcasted_iota(jnp.int32, (32, E), 0).astype(jnp.float32)
    ended = jnp.sum((tio >= end_t).astype(jnp.float32), axis=1,
                    keepdims=True)
    eid_ref[...] = jnp.minimum(ended, float(N_EXP_ - 1)).astype(jnp.int32)
    tot_ref[...] = jnp.sum(tiles, axis=1, keepdims=True).astype(jnp.int32)


def _route(x2, gate_w, gb2):
    outs = pl.pallas_call(
        _route_body,
        grid=(1,),
        in_specs=[
            pl.BlockSpec((SEQ_, D_MODEL_), lambda i: (0, 0)),
            pl.BlockSpec((N_EXP_, D_MODEL_), lambda i: (0, 0)),
            pl.BlockSpec((1, N_EXP_), lambda i: (0, 0)),
        ],
        out_specs=[
            pl.BlockSpec((SEQ_, 128), lambda i: (0, 0)),
            pl.BlockSpec((SEQ_, 128), lambda i: (0, 0)),
            pl.BlockSpec((SEQ_, 1), lambda i: (0, 0)),
            pl.BlockSpec((SEQ_, 1), lambda i: (0, 0)),
            pl.BlockSpec((32, 1), lambda i: (0, 0)),
            pl.BlockSpec((1, 1), lambda i: (0, 0)),
        ],
        out_shape=[
            jax.ShapeDtypeStruct((SEQ_, 128), jnp.float32),
            jax.ShapeDtypeStruct((SEQ_, 128), jnp.float32),
            jax.ShapeDtypeStruct((SEQ_, 1), jnp.int32),
            jax.ShapeDtypeStruct((SEQ_, 1), jnp.int32),
            jax.ShapeDtypeStruct((32, 1), jnp.int32),
            jax.ShapeDtypeStruct((1, 1), jnp.int32),
        ],
    )(x2, gate_w, gb2)
    return outs


def _dispatch_body(x_hbm, p1_hbm, p2_hbm, w1_hbm, w2_hbm,
                   xg_hbm, wg_hbm,
                   p1v, p2v, xv, w1v, w2v, sem):
    cid = lax.axis_index("c")
    sid = lax.axis_index("s")
    wid = cid * 16 + sid
    base = wid * CH
    lds = [
        pltpu.async_copy(p1_hbm.at[pl.ds(base, CH)], p1v, sem),
        pltpu.async_copy(p2_hbm.at[pl.ds(base, CH)], p2v, sem),
        pltpu.async_copy(x_hbm.at[pl.ds(base, CH)], xv, sem),
        pltpu.async_copy(w1_hbm.at[pl.ds(base, CH)], w1v, sem),
        pltpu.async_copy(w2_hbm.at[pl.ds(base, CH)], w2v, sem),
    ]
    for c in lds:
        c.wait()
    sts = [
        pltpu.async_copy(xv, xg_hbm.at[p1v], sem),
        pltpu.async_copy(xv, xg_hbm.at[p2v], sem),
        pltpu.async_copy(w1v, wg_hbm.at[p1v], sem),
        pltpu.async_copy(w2v, wg_hbm.at[p2v], sem),
    ]
    for c in sts:
        c.wait()


def _dispatch(x2, pos1, pos2, w1x, w2x):
    mesh = plsc.VectorSubcoreMesh(core_axis_name="c", subcore_axis_name="s")
    f = functools.partial(
        pl.kernel,
        mesh=mesh,
        out_type=[
            jax.ShapeDtypeStruct((T_PAD, D_MODEL_), jnp.float32),
            jax.ShapeDtypeStruct((T_PAD, 128), jnp.float32),
        ],
        scratch_types=[
            pltpu.VMEM((CH,), jnp.int32),
            pltpu.VMEM((CH,), jnp.int32),
            pltpu.VMEM((CH, D_MODEL_), jnp.float32),
            pltpu.VMEM((CH, 128), jnp.float32),
            pltpu.VMEM((CH, 128), jnp.float32),
            pltpu.SemaphoreType.DMA,
        ],
    )(_dispatch_body)
    return f(x2, pos1, pos2, w1x, w2x)


def _ffn_body(eid_ref, tot_ref, xg_ref, f1w_ref, f1b_ref, f2w_ref, f2b_ref,
              wg_ref, yg_ref):
    t = pl.program_id(0)

    @pl.when(t < tot_ref[0])
    def _():
        xb = xg_ref[...].astype(jnp.bfloat16)
        h = lax.dot_general(
            xb, f1w_ref[0], (((1,), (1,)), ((), ())),
            preferred_element_type=jnp.float32) + f1b_ref[0]
        h = _gelu_exact(h)
        y = lax.dot_general(
            h.astype(jnp.bfloat16), f2w_ref[0], (((1,), (1,)), ((), ())),
            preferred_element_type=jnp.float32) + f2b_ref[0]
        wcol = wg_ref[:, 0:1]
        yg_ref[...] = y * wcol


def _ffn(eid, tot, xg, f1w_b, fc1_b3, f2w_b, fc2_b3, wg2):
    grid_spec = pltpu.PrefetchScalarGridSpec(
        num_scalar_prefetch=2,
        grid=(T_MAX,),
        in_specs=[
            pl.BlockSpec((TM, D_MODEL_), lambda t, eid, tot: (t, 0)),
            pl.BlockSpec((1, D_FF_, D_MODEL_),
                         lambda t, eid, tot: (eid[t], 0, 0)),
            pl.BlockSpec((1, 1, D_FF_), lambda t, eid, tot: (eid[t], 0, 0)),
            pl.BlockSpec((1, D_MODEL_, D_FF_),
                         lambda t, eid, tot: (eid[t], 0, 0)),
            pl.BlockSpec((1, 1, D_MODEL_), lambda t, eid, tot: (eid[t], 0, 0)),
            pl.BlockSpec((TM, 128), lambda t, eid, tot: (t, 0)),
        ],
        out_specs=pl.BlockSpec((TM, D_MODEL_), lambda t, eid, tot: (t, 0)),
    )
    return pl.pallas_call(
        _ffn_body,
        grid_spec=grid_spec,
        out_shape=jax.ShapeDtypeStruct((T_PAD, D_MODEL_), jnp.float32),
    )(eid, tot, xg, f1w_b, fc1_b3, f2w_b, fc2_b3, wg2)


def _combine_body(yg_hbm, p1_hbm, p2_hbm, out_hbm, p1v, p2v, y1v, y2v, sem):
    cid = lax.axis_index("c")
    sid = lax.axis_index("s")
    wid = cid * 16 + sid
    base = wid * CH
    c1 = pltpu.async_copy(p1_hbm.at[pl.ds(base, CH)], p1v, sem)
    c2 = pltpu.async_copy(p2_hbm.at[pl.ds(base, CH)], p2v, sem)
    c1.wait()
    c2.wait()
    g1 = pltpu.async_copy(yg_hbm.at[p1v], y1v, sem)
    g2 = pltpu.async_copy(yg_hbm.at[p2v], y2v, sem)
    g1.wait()
    g2.wait()

    def row_body(r, carry):
        for j in range(D_MODEL_ // 16):
            sl = pl.ds(j * 16, 16)
            y1v[r, sl] = y1v[r, sl] + y2v[r, sl]
        return carry

    lax.fori_loop(0, CH, row_body, 0)
    pltpu.sync_copy(y1v, out_hbm.at[pl.ds(base, CH)])


def _combine(yg, pos1, pos2):
    mesh = plsc.VectorSubcoreMesh(core_axis_name="c", subcore_axis_name="s")
    f = functools.partial(
        pl.kernel,
        mesh=mesh,
        out_type=jax.ShapeDtypeStruct((SEQ_, D_MODEL_), jnp.float32),
        scratch_types=[
            pltpu.VMEM((CH,), jnp.int32),
            pltpu.VMEM((CH,), jnp.int32),
            pltpu.VMEM((CH, D_MODEL_), jnp.float32),
            pltpu.VMEM((CH, D_MODEL_), jnp.float32),
            pltpu.SemaphoreType.DMA,
        ],
    )(_combine_body)
    return f(yg, pos1, pos2)


def kernel(x, gate_w, gate_b, fc1_w, fc1_b, fc2_w, fc2_b):
    B, S, D = x.shape
    x2 = x.reshape(S, D)
    gb2 = gate_b.reshape(1, N_EXP_)
    w1, w2, pos1, pos2, eid, tot = _route(x2, gate_w, gb2)
    pos1 = pos1.reshape(S)
    pos2 = pos2.reshape(S)
    xg, wg = _dispatch(x2, pos1, pos2, w1, w2)
    yg = _ffn(eid.reshape(32), tot.reshape(1), xg,
              fc1_w.astype(jnp.bfloat16),
              fc1_b.reshape(N_EXP_, 1, D_FF_),
              fc2_w.astype(jnp.bfloat16),
              fc2_b.reshape(N_EXP_, 1, D_MODEL_),
              wg)
    out = _combine(yg, pos1, pos2)
    return out.reshape(B, S, D)
